# baseline (speedup 1.0000x reference)
import functools

import jax
import jax.numpy as jnp
from jax import lax
from jax.experimental import pallas as pl
from jax.experimental.pallas import tpu as pltpu
from jax.experimental.pallas import tpu_sc as plsc

NC = 2
NS = 16
NW = NC * NS
B = 128


def _sc_mesh():
    return plsc.VectorSubcoreMesh(
        core_axis_name="c", subcore_axis_name="s", num_cores=NC, num_subcores=NS
    )


NB_DEG = 8

SPLIT_NUM, SPLIT_DEN = 1, 1
NB_AGG = 2


@functools.lru_cache(maxsize=None)
def _make_deg_kernel(npad, tch):
    rpt = npad // NS
    kd = tch // NW
    ngroups = kd // NB_DEG
    assert kd % (2 * NB_DEG) == 0

    @functools.partial(
        pl.kernel,
        out_type=jax.ShapeDtypeStruct((NC, npad, 16), jnp.float32),
        mesh=_sc_mesh(),
        scratch_types=[
            [[pltpu.VMEM((B,), jnp.int32) for _ in range(NB_DEG)] for _ in range(2)],
            pltpu.VMEM((B, 16), jnp.float32),
            pltpu.VMEM_SHARED((npad, 16), jnp.float32),
            [pltpu.SemaphoreType.DMA for _ in range(2)],
            [pltpu.SemaphoreType.DMA for _ in range(NB_DEG)],
        ],
    )
    def deg_kernel(dst_hbm, ones_hbm, zeros_hbm, out_hbm,
                   dst_v, ones_v, acc_sh, isem, ssem):
        c = lax.axis_index("c")
        s = lax.axis_index("s")
        wid = s * NC + c
        base = wid * kd
        row0 = s * rpt
        pltpu.sync_copy(zeros_hbm, acc_sh.at[pl.ds(row0, rpt)])
        pltpu.sync_copy(ones_hbm, ones_v)
        plsc.subcore_barrier()

        for j in range(NB_DEG):
            pltpu.async_copy(dst_hbm.at[base + j], dst_v[0][j], isem[0])

        def group(g, p):
            for j in range(NB_DEG):
                pltpu.make_async_copy(
                    dst_hbm.at[base + g * NB_DEG + j], dst_v[p][j], isem[p]).wait()
            for j in range(NB_DEG):
                i2 = jnp.minimum(base + (g + 1) * NB_DEG + j, tch - 1)
                pltpu.async_copy(dst_hbm.at[i2], dst_v[1 - p][j], isem[1 - p])
            descs = []
            for j in range(NB_DEG):
                descs.append(pltpu.async_copy(
                    ones_v, acc_sh.at[dst_v[p][j]], ssem[j], add=True))
            for d in descs:
                d.wait()

        def body(gg, carry):
            group(gg * 2, 0)
            group(gg * 2 + 1, 1)
            return carry

        lax.fori_loop(0, ngroups // 2, body, 0)
        for j in range(NB_DEG):
            pltpu.make_async_copy(
                dst_hbm.at[tch - 1], dst_v[0][j], isem[0]).wait()
        plsc.subcore_barrier()
        pltpu.sync_copy(acc_sh.at[pl.ds(row0, rpt)], out_hbm.at[c, pl.ds(row0, rpt)])

    return deg_kernel


@functools.lru_cache(maxsize=None)
def _make_agg_kernel(npad, tch, k0):
    rpt = npad // NS
    k1 = tch // NS - k0
    assert k0 % (2 * NB_AGG) == 0 and k1 % (2 * NB_AGG) == 0 and k1 >= 0

    @functools.partial(
        pl.kernel,
        out_type=jax.ShapeDtypeStruct((NC, npad, 128), jnp.float32),
        mesh=_sc_mesh(),
        scratch_types=[
            [[pltpu.VMEM((B,), jnp.int32) for _ in range(NB_AGG)] for _ in range(2)],
            [[pltpu.VMEM((B,), jnp.int32) for _ in range(NB_AGG)] for _ in range(2)],
            [pltpu.VMEM((B, 128), jnp.float32) for _ in range(NB_AGG)],
            pltpu.VMEM_SHARED((npad, 128), jnp.float32),
            [pltpu.SemaphoreType.DMA for _ in range(2)],
            [pltpu.SemaphoreType.DMA for _ in range(NB_AGG)],
            [pltpu.SemaphoreType.DMA for _ in range(NB_AGG)],
        ],
    )
    def agg_kernel(y_hbm, src_hbm, dst_hbm, zeros_hbm, out_hbm,
                   src_v, dst_v, rows, acc_sh, isem, gsem, ssem):
        c = lax.axis_index("c")
        s = lax.axis_index("s")
        row0 = s * rpt
        base = jnp.where(c == 0, s * k0, NS * k0 + s * k1)
        ngg = jnp.where(c == 0, k0 // (2 * NB_AGG), k1 // (2 * NB_AGG))
        pltpu.sync_copy(zeros_hbm, acc_sh.at[pl.ds(row0, rpt)])
        plsc.subcore_barrier()

        for j in range(NB_AGG):
            i0 = jnp.minimum(base + j, tch - 1)
            pltpu.async_copy(src_hbm.at[i0], src_v[0][j], isem[0])
            pltpu.async_copy(dst_hbm.at[i0], dst_v[0][j], isem[0])

        def group(g, p):
            for j in range(NB_AGG):
                i = base + g * NB_AGG + j
                pltpu.make_async_copy(src_hbm.at[i], src_v[p][j], isem[p]).wait()
                pltpu.make_async_copy(dst_hbm.at[i], dst_v[p][j], isem[p]).wait()
            gdescs = []
            for j in range(NB_AGG):
                gdescs.append(pltpu.async_copy(
                    y_hbm.at[src_v[p][j]], rows[j], gsem[j]))
            for j in range(NB_AGG):
                i2 = jnp.minimum(base + (g + 1) * NB_AGG + j, tch - 1)
                pltpu.async_copy(src_hbm.at[i2], src_v[1 - p][j], isem[1 - p])
                pltpu.async_copy(dst_hbm.at[i2], dst_v[1 - p][j], isem[1 - p])
            sdescs = []
            for j in range(NB_AGG):
                gdescs[j].wait()
                sdescs.append(pltpu.async_copy(
                    rows[j], acc_sh.at[dst_v[p][j]], ssem[j], add=True))
            for d in sdescs:
                d.wait()

        def body(gg, carry):
            group(gg * 2, 0)
            group(gg * 2 + 1, 1)
            return carry

        lax.fori_loop(0, ngg, body, 0)
        for j in range(NB_AGG):
            pltpu.make_async_copy(src_hbm.at[tch - 1], src_v[0][j], isem[0]).wait()
            pltpu.make_async_copy(dst_hbm.at[tch - 1], dst_v[0][j], isem[0]).wait()
        plsc.subcore_barrier()
        pltpu.sync_copy(acc_sh.at[pl.ds(row0, rpt)], out_hbm.at[c, pl.ds(row0, rpt)])

    return agg_kernel


def _tc1_body(x_ref, degp_ref, w1_ref, y_ref, dinv_ref):
    dp = degp_ref[...]
    deg = dp[0, :, 0:1] + dp[1, :, 0:1] + 1.0
    dinv = lax.rsqrt(deg)
    xw = jnp.dot(x_ref[...], w1_ref[...], preferred_element_type=jnp.float32)
    y_ref[...] = xw * dinv
    dinv_ref[...] = dinv


def _tc2_body(y1_ref, p_ref, dinv_ref, b1_ref, w2_ref, y2_ref):
    pr = p_ref[...]
    dinv = dinv_ref[...]
    h = jnp.maximum(dinv * (pr[0] + pr[1] + y1_ref[...]) + b1_ref[...], 0.0)
    y2_ref[...] = jnp.dot(h, w2_ref[...], preferred_element_type=jnp.float32) * dinv


def _tc3_body(y2_ref, q_ref, dinv_ref, b2_ref, wout_ref, bout_ref, o_ref):
    qr = q_ref[...]
    h = jnp.maximum(dinv_ref[...] * (qr[0] + qr[1] + y2_ref[...]) + b2_ref[...], 0.0)
    o_ref[...] = jnp.dot(h, wout_ref[...], preferred_element_type=jnp.float32) + bout_ref[...]


def kernel(x, edge_index, W1, b1, W2, b2, Wout, bout):
    n, d = x.shape
    h_dim = W1.shape[1]
    c_dim = Wout.shape[1]
    e = edge_index.shape[1]

    npad = -(-(n + 1) // (NS * 8)) * (NS * 8)
    tch = -(-e // B)
    tch = -(-tch // (NW * 16)) * (NW * 16)
    ep = tch * B
    rpt = npad // NS

    src = edge_index[0]
    dst = edge_index[1]
    pad = ep - e
    if pad:
        src = jnp.concatenate([src, jnp.zeros((pad,), src.dtype)])
        dst = jnp.concatenate([dst, jnp.full((pad,), n, dst.dtype)])
    src = src.astype(jnp.int32).reshape(tch, B)
    dst = dst.astype(jnp.int32).reshape(tch, B)

    ones16 = jnp.ones((B, 16), jnp.float32)
    zeros16 = jnp.zeros((rpt, 16), jnp.float32)
    zeros128 = jnp.zeros((rpt, 128), jnp.float32)

    k0 = (tch // NS * SPLIT_NUM // SPLIT_DEN) // (2 * NB_AGG) * (2 * NB_AGG)
    deg_k = _make_deg_kernel(npad, tch)
    agg_k = _make_agg_kernel(npad, tch, k0)

    degp = deg_k(dst, ones16, zeros16)

    r = 2000
    grid = (n // r,)
    bcast = lambda i: (0, 0)
    row_im = lambda i: (i, 0)
    part_im = lambda i: (0, i, 0)

    y1, dinv = pl.pallas_call(
        _tc1_body,
        grid=grid,
        in_specs=[
            pl.BlockSpec((r, d), row_im),
            pl.BlockSpec((NC, r, 16), part_im),
            pl.BlockSpec((d, h_dim), bcast),
        ],
        out_specs=[
            pl.BlockSpec((r, h_dim), row_im),
            pl.BlockSpec((r, 1), row_im),
        ],
        out_shape=[
            jax.ShapeDtypeStruct((n, h_dim), jnp.float32),
            jax.ShapeDtypeStruct((n, 1), jnp.float32),
        ],
    )(x, degp, W1)

    p1 = agg_k(y1, src, dst, zeros128)

    y2 = pl.pallas_call(
        _tc2_body,
        grid=grid,
        in_specs=[
            pl.BlockSpec((r, h_dim), row_im),
            pl.BlockSpec((NC, r, h_dim), part_im),
            pl.BlockSpec((r, 1), row_im),
            pl.BlockSpec((1, h_dim), bcast),
            pl.BlockSpec((h_dim, h_dim), bcast),
        ],
        out_specs=pl.BlockSpec((r, h_dim), row_im),
        out_shape=jax.ShapeDtypeStruct((n, h_dim), jnp.float32),
    )(y1, p1, dinv, b1.reshape(1, -1), W2)

    p2 = agg_k(y2, src, dst, zeros128)

    out = pl.pallas_call(
        _tc3_body,
        grid=grid,
        in_specs=[
            pl.BlockSpec((r, h_dim), row_im),
            pl.BlockSpec((NC, r, h_dim), part_im),
            pl.BlockSpec((r, 1), row_im),
            pl.BlockSpec((1, h_dim), bcast),
            pl.BlockSpec((h_dim, c_dim), bcast),
            pl.BlockSpec((1, c_dim), bcast),
        ],
        out_specs=pl.BlockSpec((r, c_dim), row_im),
        out_shape=jax.ShapeDtypeStruct((n, c_dim), jnp.float32),
    )(y2, p2, dinv, b2.reshape(1, -1), Wout, bout.reshape(1, -1))

    return out


# speedup vs baseline: 1.1418x; 1.1418x over previous
import functools

import jax
import jax.numpy as jnp
from jax import lax
from jax.experimental import pallas as pl
from jax.experimental.pallas import tpu as pltpu
from jax.experimental.pallas import tpu_sc as plsc

NC = 2
NS = 16
NW = NC * NS
B = 128


def _sc_mesh():
    return plsc.VectorSubcoreMesh(
        core_axis_name="c", subcore_axis_name="s", num_cores=NC, num_subcores=NS
    )


@functools.lru_cache(maxsize=None)
def _make_deg_kernel(npad, ept):
    rpt = npad // NS
    nchunks = ept // B

    @functools.partial(
        pl.kernel,
        out_type=jax.ShapeDtypeStruct((NC, npad, 16), jnp.float32),
        mesh=_sc_mesh(),
        scratch_types=[
            pltpu.VMEM((B,), jnp.int32),
            pltpu.VMEM((B, 16), jnp.float32),
            pltpu.VMEM_SHARED((npad, 16), jnp.float32),
        ],
    )
    def deg_kernel(dst_hbm, ones_hbm, zeros_hbm, out_hbm, dst_v, ones_v, acc_sh):
        c = lax.axis_index("c")
        s = lax.axis_index("s")
        wid = s * NC + c
        row0 = s * rpt
        pltpu.sync_copy(zeros_hbm, acc_sh.at[pl.ds(row0, rpt)])
        pltpu.sync_copy(ones_hbm, ones_v)
        plsc.subcore_barrier()
        base = wid * ept

        def body(i, carry):
            pltpu.sync_copy(dst_hbm.at[pl.ds(base + i * B, B)], dst_v)
            pltpu.sync_copy(ones_v, acc_sh.at[dst_v], add=True)
            return carry

        lax.fori_loop(0, nchunks, body, 0)
        plsc.subcore_barrier()
        pltpu.sync_copy(acc_sh.at[pl.ds(row0, rpt)], out_hbm.at[c, pl.ds(row0, rpt)])

    return deg_kernel


@functools.lru_cache(maxsize=None)
def _make_agg_kernel(npad, ept):
    rpt = npad // NS
    nchunks = ept // B

    @functools.partial(
        pl.kernel,
        out_type=jax.ShapeDtypeStruct((NC, npad, 128), jnp.float32),
        mesh=_sc_mesh(),
        scratch_types=[
            pltpu.VMEM((B,), jnp.int32),
            pltpu.VMEM((B,), jnp.int32),
            pltpu.VMEM((B, 128), jnp.float32),
            pltpu.VMEM_SHARED((npad, 128), jnp.float32),
            pltpu.SemaphoreType.DMA,
        ],
    )
    def agg_kernel(y_hbm, src_hbm, dst_hbm, zeros_hbm, out_hbm,
                   src_v, dst_v, rows_v, acc_sh, sem):
        c = lax.axis_index("c")
        s = lax.axis_index("s")
        wid = s * NC + c
        row0 = s * rpt
        pltpu.sync_copy(zeros_hbm, acc_sh.at[pl.ds(row0, rpt)])
        plsc.subcore_barrier()
        base = wid * ept

        def body(i, carry):
            off = base + i * B
            pltpu.sync_copy(src_hbm.at[pl.ds(off, B)], src_v)
            pltpu.sync_copy(dst_hbm.at[pl.ds(off, B)], dst_v)
            pltpu.async_copy(y_hbm.at[src_v], rows_v, sem).wait()
            pltpu.sync_copy(rows_v, acc_sh.at[dst_v], add=True)
            return carry

        lax.fori_loop(0, nchunks, body, 0)
        plsc.subcore_barrier()
        pltpu.sync_copy(acc_sh.at[pl.ds(row0, rpt)], out_hbm.at[c, pl.ds(row0, rpt)])

    return agg_kernel


def _tc1_body(x_ref, degp_ref, w1_ref, y_ref, dinv_ref):
    dp = degp_ref[...]
    deg = dp[0, :, 0:1] + dp[1, :, 0:1] + 1.0
    dinv = lax.rsqrt(deg)
    xw = jnp.dot(x_ref[...], w1_ref[...], preferred_element_type=jnp.float32)
    y_ref[...] = xw * dinv
    dinv_ref[...] = dinv


def _tc2_body(y1_ref, p_ref, dinv_ref, b1_ref, w2_ref, y2_ref):
    pr = p_ref[...]
    dinv = dinv_ref[...]
    h = jnp.maximum(dinv * (pr[0] + pr[1] + y1_ref[...]) + b1_ref[...], 0.0)
    y2_ref[...] = jnp.dot(h, w2_ref[...], preferred_element_type=jnp.float32) * dinv


def _tc3_body(y2_ref, q_ref, dinv_ref, b2_ref, wout_ref, bout_ref, o_ref):
    qr = q_ref[...]
    h = jnp.maximum(dinv_ref[...] * (qr[0] + qr[1] + y2_ref[...]) + b2_ref[...], 0.0)
    o_ref[...] = jnp.dot(h, wout_ref[...], preferred_element_type=jnp.float32) + bout_ref[...]


def kernel(x, edge_index, W1, b1, W2, b2, Wout, bout):
    n, d = x.shape
    h_dim = W1.shape[1]
    c_dim = Wout.shape[1]
    e = edge_index.shape[1]

    npad = -(-(n + 1) // (NS * 8)) * (NS * 8)
    nchunks = -(-e // (NW * B))
    nchunks = -(-nchunks // 16) * 16
    ep = nchunks * NW * B
    ept = nchunks * B
    rpt = npad // NS

    src = edge_index[0]
    dst = edge_index[1]
    pad = ep - e
    if pad:
        src = jnp.concatenate([src, jnp.zeros((pad,), src.dtype)])
        dst = jnp.concatenate([dst, jnp.full((pad,), n, dst.dtype)])
    src = src.astype(jnp.int32)
    dst = dst.astype(jnp.int32)

    ones16 = jnp.ones((B, 16), jnp.float32)
    zeros16 = jnp.zeros((rpt, 16), jnp.float32)
    zeros128 = jnp.zeros((rpt, 128), jnp.float32)

    deg_k = _make_deg_kernel(npad, ept)
    agg_k = _make_agg_kernel(npad, ept)

    degp = deg_k(dst, ones16, zeros16)

    r = 2000
    grid = (n // r,)
    bcast = lambda i: (0, 0)
    row_im = lambda i: (i, 0)
    part_im = lambda i: (0, i, 0)

    y1, dinv = pl.pallas_call(
        _tc1_body,
        grid=grid,
        in_specs=[
            pl.BlockSpec((r, d), row_im),
            pl.BlockSpec((NC, r, 16), part_im),
            pl.BlockSpec((d, h_dim), bcast),
        ],
        out_specs=[
            pl.BlockSpec((r, h_dim), row_im),
            pl.BlockSpec((r, 1), row_im),
        ],
        out_shape=[
            jax.ShapeDtypeStruct((n, h_dim), jnp.float32),
            jax.ShapeDtypeStruct((n, 1), jnp.float32),
        ],
    )(x, degp, W1)

    p1 = agg_k(y1, src, dst, zeros128)

    y2 = pl.pallas_call(
        _tc2_body,
        grid=grid,
        in_specs=[
            pl.BlockSpec((r, h_dim), row_im),
            pl.BlockSpec((NC, r, h_dim), part_im),
            pl.BlockSpec((r, 1), row_im),
            pl.BlockSpec((1, h_dim), bcast),
            pl.BlockSpec((h_dim, h_dim), bcast),
        ],
        out_specs=pl.BlockSpec((r, h_dim), row_im),
        out_shape=jax.ShapeDtypeStruct((n, h_dim), jnp.float32),
    )(y1, p1, dinv, b1.reshape(1, -1), W2)

    p2 = agg_k(y2, src, dst, zeros128)

    out = pl.pallas_call(
        _tc3_body,
        grid=grid,
        in_specs=[
            pl.BlockSpec((r, h_dim), row_im),
            pl.BlockSpec((NC, r, h_dim), part_im),
            pl.BlockSpec((r, 1), row_im),
            pl.BlockSpec((1, h_dim), bcast),
            pl.BlockSpec((h_dim, c_dim), bcast),
            pl.BlockSpec((1, c_dim), bcast),
        ],
        out_specs=pl.BlockSpec((r, c_dim), row_im),
        out_shape=jax.ShapeDtypeStruct((n, c_dim), jnp.float32),
    )(y2, p2, dinv, b2.reshape(1, -1), Wout, bout.reshape(1, -1))

    return out


# speedup vs baseline: 1.4937x; 1.3082x over previous
import functools

import jax
import jax.numpy as jnp
from jax import lax
from jax.experimental import pallas as pl
from jax.experimental.pallas import tpu as pltpu
from jax.experimental.pallas import tpu_sc as plsc

NC = 2
NS = 16
NW = NC * NS
B = 128


def _sc_mesh():
    return plsc.VectorSubcoreMesh(
        core_axis_name="c", subcore_axis_name="s", num_cores=NC, num_subcores=NS
    )


@functools.lru_cache(maxsize=None)
def _make_deg_kernel(npad, ept):
    rpt = npad // NS
    nchunks = ept // B

    @functools.partial(
        pl.kernel,
        out_type=jax.ShapeDtypeStruct((NC, npad, 16), jnp.float32),
        mesh=_sc_mesh(),
        scratch_types=[
            pltpu.VMEM((B,), jnp.int32),
            pltpu.VMEM((B, 16), jnp.float32),
            pltpu.VMEM_SHARED((npad, 16), jnp.float32),
        ],
    )
    def deg_kernel(dst_hbm, ones_hbm, zeros_hbm, out_hbm, dst_v, ones_v, acc_sh):
        c = lax.axis_index("c")
        s = lax.axis_index("s")
        wid = s * NC + c
        row0 = s * rpt
        pltpu.sync_copy(zeros_hbm, acc_sh.at[pl.ds(row0, rpt)])
        pltpu.sync_copy(ones_hbm, ones_v)
        plsc.subcore_barrier()
        base = wid * ept

        def body(i, carry):
            pltpu.sync_copy(dst_hbm.at[pl.ds(base + i * B, B)], dst_v)
            pltpu.sync_copy(ones_v, acc_sh.at[dst_v], add=True)
            return carry

        lax.fori_loop(0, nchunks, body, 0)
        plsc.subcore_barrier()
        pltpu.sync_copy(acc_sh.at[pl.ds(row0, rpt)], out_hbm.at[c, pl.ds(row0, rpt)])

    return deg_kernel


@functools.lru_cache(maxsize=None)
def _make_agg_kernel(npad, ept):
    rpt = npad // NS
    nchunks = ept // B

    @functools.partial(
        pl.kernel,
        out_type=jax.ShapeDtypeStruct((NC, npad, 128), jnp.float32),
        mesh=_sc_mesh(),
        scratch_types=[
            pltpu.VMEM((B,), jnp.int32),
            pltpu.VMEM((B,), jnp.int32),
            pltpu.VMEM((B, 128), jnp.float32),
            pltpu.VMEM_SHARED((npad, 128), jnp.float32),
            pltpu.SemaphoreType.DMA,
        ],
    )
    def agg_kernel(y_hbm, src_hbm, dst_hbm, zeros_hbm, out_hbm,
                   src_v, dst_v, rows_v, acc_sh, sem):
        c = lax.axis_index("c")
        s = lax.axis_index("s")
        wid = s * NC + c
        row0 = s * rpt
        pltpu.sync_copy(zeros_hbm, acc_sh.at[pl.ds(row0, rpt)])
        plsc.subcore_barrier()
        base = wid * ept

        def body(i, carry):
            off = base + i * B
            pltpu.sync_copy(src_hbm.at[pl.ds(off, B)], src_v)
            pltpu.sync_copy(dst_hbm.at[pl.ds(off, B)], dst_v)
            pltpu.async_copy(y_hbm.at[src_v], rows_v, sem).wait()
            pltpu.sync_copy(rows_v, acc_sh.at[dst_v], add=True)
            return carry

        lax.fori_loop(0, nchunks, body, 0)
        plsc.subcore_barrier()
        pltpu.sync_copy(acc_sh.at[pl.ds(row0, rpt)], out_hbm.at[c, pl.ds(row0, rpt)])

    return agg_kernel


def _tc1_body(x_ref, degp_ref, w1_ref, y_ref, dinv_ref):
    dp = degp_ref[...]
    deg = dp[0, :, 0:1] + dp[1, :, 0:1] + 1.0
    dinv = lax.rsqrt(deg)
    xw = jnp.dot(x_ref[...], w1_ref[...], preferred_element_type=jnp.float32)
    y_ref[...] = xw * dinv
    dinv_ref[...] = dinv


def _tc2_body(y1_ref, p_ref, dinv_ref, b1_ref, w2_ref, y2_ref):
    pr = p_ref[...]
    dinv = dinv_ref[...]
    h = jnp.maximum(dinv * (pr[0] + pr[1] + y1_ref[...]) + b1_ref[...], 0.0)
    y2_ref[...] = jnp.dot(h, w2_ref[...], preferred_element_type=jnp.float32) * dinv


def _tc3_body(y2_ref, q_ref, dinv_ref, b2_ref, wout_ref, bout_ref, o_ref):
    qr = q_ref[...]
    h = jnp.maximum(dinv_ref[...] * (qr[0] + qr[1] + y2_ref[...]) + b2_ref[...], 0.0)
    o_ref[...] = jnp.dot(h, wout_ref[...], preferred_element_type=jnp.float32) + bout_ref[...]


def kernel(x, edge_index, W1, b1, W2, b2, Wout, bout):
    n, d = x.shape
    h_dim = W1.shape[1]
    c_dim = Wout.shape[1]
    e = edge_index.shape[1]

    npad = -(-(n + 1) // (NS * 8)) * (NS * 8)
    nchunks = -(-e // (NW * B))
    ep = nchunks * NW * B
    ept = nchunks * B
    rpt = npad // NS

    src = edge_index[0]
    dst = edge_index[1]
    pad = ep - e
    if pad:
        src = jnp.concatenate([src, jnp.zeros((pad,), src.dtype)])
        dst = jnp.concatenate([dst, jnp.full((pad,), n, dst.dtype)])
    src = src.astype(jnp.int32)
    dst = dst.astype(jnp.int32)

    ones16 = jnp.ones((B, 16), jnp.float32)
    zeros16 = jnp.zeros((rpt, 16), jnp.float32)
    zeros128 = jnp.zeros((rpt, 128), jnp.float32)

    deg_k = _make_deg_kernel(npad, ept)
    agg_k = _make_agg_kernel(npad, ept)

    degp = deg_k(dst, ones16, zeros16)

    r = 2000
    grid = (n // r,)
    bcast = lambda i: (0, 0)
    row_im = lambda i: (i, 0)
    part_im = lambda i: (0, i, 0)

    y1, dinv = pl.pallas_call(
        _tc1_body,
        grid=grid,
        in_specs=[
            pl.BlockSpec((r, d), row_im),
            pl.BlockSpec((NC, r, 16), part_im),
            pl.BlockSpec((d, h_dim), bcast),
        ],
        out_specs=[
            pl.BlockSpec((r, h_dim), row_im),
            pl.BlockSpec((r, 1), row_im),
        ],
        out_shape=[
            jax.ShapeDtypeStruct((n, h_dim), jnp.float32),
            jax.ShapeDtypeStruct((n, 1), jnp.float32),
        ],
    )(x, degp, W1)

    p1 = agg_k(y1, src, dst, zeros128)

    y2 = pl.pallas_call(
        _tc2_body,
        grid=grid,
        in_specs=[
            pl.BlockSpec((r, h_dim), row_im),
            pl.BlockSpec((NC, r, h_dim), part_im),
            pl.BlockSpec((r, 1), row_im),
            pl.BlockSpec((1, h_dim), bcast),
            pl.BlockSpec((h_dim, h_dim), bcast),
        ],
        out_specs=pl.BlockSpec((r, h_dim), row_im),
        out_shape=jax.ShapeDtypeStruct((n, h_dim), jnp.float32),
    )(y1, p1, dinv, b1.reshape(1, -1), W2)

    p2 = agg_k(y2, src, dst, zeros128)

    out = pl.pallas_call(
        _tc3_body,
        grid=grid,
        in_specs=[
            pl.BlockSpec((r, h_dim), row_im),
            pl.BlockSpec((NC, r, h_dim), part_im),
            pl.BlockSpec((r, 1), row_im),
            pl.BlockSpec((1, h_dim), bcast),
            pl.BlockSpec((h_dim, c_dim), bcast),
            pl.BlockSpec((1, c_dim), bcast),
        ],
        out_specs=pl.BlockSpec((r, c_dim), row_im),
        out_shape=jax.ShapeDtypeStruct((n, c_dim), jnp.float32),
    )(y2, p2, dinv, b2.reshape(1, -1), Wout, bout.reshape(1, -1))

    return out
